# Initial kernel scaffold; baseline (speedup 1.0000x reference)
import jax
import jax.numpy as jnp
from jax.experimental import pallas as pl


def kernel(x, edge_index, W_l1, b_l1, W_r1, W_l2, b_l2, W_r2, W_c, b_c):
    raise NotImplementedError("write your pallas kernel here")


# speedup vs baseline: 6.0241x; 6.0241x over previous
import functools

import jax
import jax.numpy as jnp
from jax import lax
from jax.experimental import pallas as pl
from jax.experimental.pallas import tpu as pltpu
from jax.experimental.pallas import tpu_sc as plsc

N = 10000
E = 320000
IN_D = 128
D = 64
CW = 16

NC, NS = 2, 16
NW = NC * NS
EPW = E // NW
CH = 80
NCHUNK = EPW // CH
NP = 10240
RPT = NP // NS

RB = 2000
GRID = N // RB

@functools.cache
def _mesh():
    return plsc.VectorSubcoreMesh(core_axis_name="c", subcore_axis_name="s")


def _sc_agg_body(with_cnt, *refs):
    if with_cnt:
        (table, src, dst, zrow, zcnt, ones,
         out_sum, out_cnt, src_v, dst_v, rows_v, ones_v,
         acc_sh, cnt_sh, sem) = refs
    else:
        (table, src, dst, zrow,
         out_sum, src_v, dst_v, rows_v,
         acc_sh, sem) = refs
    cid = lax.axis_index("c")
    sid = lax.axis_index("s")
    wid = cid * NS + sid
    r0 = sid * RPT
    pltpu.sync_copy(zrow, acc_sh.at[pl.ds(r0, RPT)])
    if with_cnt:
        pltpu.sync_copy(zcnt, cnt_sh.at[pl.ds(r0, RPT)])
        pltpu.sync_copy(ones, ones_v)
    plsc.subcore_barrier()

    def body(i, carry):
        base = pl.multiple_of(wid * EPW + i * CH, 8)
        pltpu.sync_copy(src.at[pl.ds(base, CH)], src_v)
        pltpu.sync_copy(dst.at[pl.ds(base, CH)], dst_v)
        pltpu.async_copy(table.at[src_v], rows_v, sem).wait()
        pltpu.sync_copy(rows_v, acc_sh.at[dst_v], add=True)
        if with_cnt:
            pltpu.sync_copy(ones_v, cnt_sh.at[dst_v], add=True)
        return carry

    lax.fori_loop(0, NCHUNK, body, 0)
    plsc.subcore_barrier()
    pltpu.sync_copy(acc_sh.at[pl.ds(r0, RPT)], out_sum.at[cid].at[pl.ds(r0, RPT)])
    if with_cnt:
        pltpu.sync_copy(cnt_sh.at[pl.ds(r0, RPT)], out_cnt.at[cid].at[pl.ds(r0, RPT)])


@functools.cache
def _sc_agg1():
    return functools.partial(
        pl.kernel,
        mesh=_mesh(),
        compiler_params=pltpu.CompilerParams(use_tc_tiling_on_sc=False),
        out_type=(jax.ShapeDtypeStruct((NC, NP, D), jnp.float32),
                  jax.ShapeDtypeStruct((NC, NP, CW), jnp.float32)),
        scratch_types=[
            pltpu.VMEM((CH,), jnp.int32),
            pltpu.VMEM((CH,), jnp.int32),
            pltpu.VMEM((CH, D), jnp.float32),
            pltpu.VMEM((CH, CW), jnp.float32),
            pltpu.VMEM_SHARED((NP, D), jnp.float32),
            pltpu.VMEM_SHARED((NP, CW), jnp.float32),
            pltpu.SemaphoreType.DMA,
        ],
    )(functools.partial(_sc_agg_body, True))


@functools.cache
def _sc_agg2():
    return functools.partial(
        pl.kernel,
        mesh=_mesh(),
        compiler_params=pltpu.CompilerParams(use_tc_tiling_on_sc=False),
        out_type=jax.ShapeDtypeStruct((NC, NP, D), jnp.float32),
        scratch_types=[
            pltpu.VMEM((CH,), jnp.int32),
            pltpu.VMEM((CH,), jnp.int32),
            pltpu.VMEM((CH, D), jnp.float32),
            pltpu.VMEM_SHARED((NP, D), jnp.float32),
            pltpu.SemaphoreType.DMA,
        ],
    )(functools.partial(_sc_agg_body, False))


def _proj_body(x_ref, wl_ref, wr_ref, p_ref, xr_ref):
    xv = x_ref[...]
    p_ref[...] = jnp.dot(xv, wl_ref[...], preferred_element_type=jnp.float32)
    xr_ref[...] = jnp.dot(xv, wr_ref[...], preferred_element_type=jnp.float32)


_tc_proj = pl.pallas_call(
    _proj_body,
    grid=(GRID,),
    in_specs=[
        pl.BlockSpec((RB, IN_D), lambda i: (i, 0)),
        pl.BlockSpec((IN_D, D), lambda i: (0, 0)),
        pl.BlockSpec((IN_D, D), lambda i: (0, 0)),
    ],
    out_specs=[
        pl.BlockSpec((RB, D), lambda i: (i, 0)),
        pl.BlockSpec((RB, D), lambda i: (i, 0)),
    ],
    out_shape=[jax.ShapeDtypeStruct((N, D), jnp.float32)] * 2,
)


def _mid_body(s_ref, c_ref, xr_ref, b_ref, wl2_ref, wr2_ref, p2_ref, hr2_ref):
    ssum = s_ref[0] + s_ref[1]
    cnt = c_ref[0, :, 0:1] + c_ref[1, :, 0:1]
    inv = 1.0 / jnp.maximum(cnt, 1.0)
    h = jnp.maximum(ssum * inv + b_ref[...] + xr_ref[...], 0.0)
    p2_ref[...] = jnp.dot(h, wl2_ref[...], preferred_element_type=jnp.float32)
    hr2_ref[...] = jnp.dot(h, wr2_ref[...], preferred_element_type=jnp.float32)


_tc_mid = pl.pallas_call(
    _mid_body,
    grid=(GRID,),
    in_specs=[
        pl.BlockSpec((NC, RB, D), lambda i: (0, i, 0)),
        pl.BlockSpec((NC, RB, CW), lambda i: (0, i, 0)),
        pl.BlockSpec((RB, D), lambda i: (i, 0)),
        pl.BlockSpec((1, D), lambda i: (0, 0)),
        pl.BlockSpec((D, D), lambda i: (0, 0)),
        pl.BlockSpec((D, D), lambda i: (0, 0)),
    ],
    out_specs=[
        pl.BlockSpec((RB, D), lambda i: (i, 0)),
        pl.BlockSpec((RB, D), lambda i: (i, 0)),
    ],
    out_shape=[jax.ShapeDtypeStruct((N, D), jnp.float32)] * 2,
)


def _out_body(s_ref, c_ref, hr_ref, b_ref, wc_ref, bc_ref, o_ref):
    ssum = s_ref[0] + s_ref[1]
    cnt = c_ref[0, :, 0:1] + c_ref[1, :, 0:1]
    inv = 1.0 / jnp.maximum(cnt, 1.0)
    h2 = jnp.maximum(ssum * inv + b_ref[...] + hr_ref[...], 0.0)
    o_ref[...] = jnp.dot(h2, wc_ref[...], preferred_element_type=jnp.float32) + bc_ref[...]


def _make_tc_out(out_dim):
    return pl.pallas_call(
        _out_body,
        grid=(GRID,),
        in_specs=[
            pl.BlockSpec((NC, RB, D), lambda i: (0, i, 0)),
            pl.BlockSpec((NC, RB, CW), lambda i: (0, i, 0)),
            pl.BlockSpec((RB, D), lambda i: (i, 0)),
            pl.BlockSpec((1, D), lambda i: (0, 0)),
            pl.BlockSpec((D, out_dim), lambda i: (0, 0)),
            pl.BlockSpec((1, out_dim), lambda i: (0, 0)),
        ],
        out_specs=pl.BlockSpec((RB, out_dim), lambda i: (i, 0)),
        out_shape=jax.ShapeDtypeStruct((N, out_dim), jnp.float32),
    )


def kernel(x, edge_index, W_l1, b_l1, W_r1, W_l2, b_l2, W_r2, W_c, b_c):
    src = edge_index[0].astype(jnp.int32)
    dst = edge_index[1].astype(jnp.int32)
    zrow = jnp.zeros((RPT, D), jnp.float32)
    zcnt = jnp.zeros((RPT, CW), jnp.float32)
    ones = jnp.ones((CH, CW), jnp.float32)

    p1, xr1 = _tc_proj(x, W_l1, W_r1)
    s1, c1 = _sc_agg1()(p1, src, dst, zrow, zcnt, ones)
    p2, hr2 = _tc_mid(s1, c1, xr1, b_l1.reshape(1, -1), W_l2, W_r2)
    s2 = _sc_agg2()(p2, src, dst, zrow)
    out = _make_tc_out(W_c.shape[1])(s2, c1, hr2, b_l2.reshape(1, -1),
                                     W_c, b_c.reshape(1, -1))
    return out


# speedup vs baseline: 13.6416x; 2.2645x over previous
import functools

import jax
import jax.numpy as jnp
from jax import lax
from jax.experimental import pallas as pl
from jax.experimental.pallas import tpu as pltpu
from jax.experimental.pallas import tpu_sc as plsc

N = 10000
E = 320000
IN_D = 128
D = 64
CW = 16

NC, NS = 2, 16
NW = NC * NS
EPW = E // NW
CH = 80
NCHUNK = EPW // CH
NP = 10240
RPT = NP // NS

RB = 2000
GRID = N // RB

@functools.cache
def _mesh():
    return plsc.VectorSubcoreMesh(core_axis_name="c", subcore_axis_name="s")


def _sc_agg_body(with_cnt, *refs):
    if with_cnt:
        (table, src3, dst3, zrow, zcnt, ones,
         out_sum, out_cnt,
         srcs_v, dsts_v, rows0, rows1, ones_v,
         acc_sh, cnt_sh,
         sg0, sg1, ss0, ss1, sc0, sc1) = refs
    else:
        (table, src3, dst3, zrow,
         out_sum,
         srcs_v, dsts_v, rows0, rows1,
         acc_sh,
         sg0, sg1, ss0, ss1) = refs
    cid = lax.axis_index("c")
    sid = lax.axis_index("s")
    wid = cid * NS + sid
    r0 = sid * RPT
    pltpu.sync_copy(zrow, acc_sh.at[pl.ds(r0, RPT)])
    if with_cnt:
        pltpu.sync_copy(zcnt, cnt_sh.at[pl.ds(r0, RPT)])
        pltpu.sync_copy(ones, ones_v)
    pltpu.sync_copy(src3.at[wid], srcs_v)
    pltpu.sync_copy(dst3.at[wid], dsts_v)
    plsc.subcore_barrier()

    rows = (rows0, rows1)
    sg = (sg0, sg1)
    ss = (ss0, ss1)
    scs = (sc0, sc1) if with_cnt else None

    def wait_scatter(b, c):
        pltpu.make_async_copy(rows[b], acc_sh.at[dsts_v.at[c]], ss[b]).wait()
        if with_cnt:
            pltpu.make_async_copy(ones_v, cnt_sh.at[dsts_v.at[c]], scs[b]).wait()

    pltpu.async_copy(table.at[srcs_v.at[0]], rows0, sg0)

    def pair(g, carry):
        for b in (0, 1):
            c = g * 2 + b
            bo = 1 - b
            if b == 1:
                wait_scatter(bo, c)
            else:
                @pl.when(g > 0)
                def _():
                    wait_scatter(bo, c)
            pltpu.async_copy(table.at[srcs_v.at[c + 1]], rows[bo], sg[bo])
            pltpu.make_async_copy(table.at[srcs_v.at[c]], rows[b], sg[b]).wait()
            pltpu.async_copy(rows[b], acc_sh.at[dsts_v.at[c]], ss[b], add=True)
            if with_cnt:
                pltpu.async_copy(ones_v, cnt_sh.at[dsts_v.at[c]], scs[b], add=True)
        return carry

    lax.fori_loop(0, (NCHUNK - 1) // 2, pair, 0)
    cl = NCHUNK - 1
    wait_scatter(1, cl)
    pltpu.make_async_copy(table.at[srcs_v.at[cl]], rows0, sg0).wait()
    pltpu.sync_copy(rows0, acc_sh.at[dsts_v.at[cl]], add=True)
    if with_cnt:
        pltpu.sync_copy(ones_v, cnt_sh.at[dsts_v.at[cl]], add=True)
    plsc.subcore_barrier()
    pltpu.sync_copy(acc_sh.at[pl.ds(r0, RPT)], out_sum.at[cid].at[pl.ds(r0, RPT)])
    if with_cnt:
        pltpu.sync_copy(cnt_sh.at[pl.ds(r0, RPT)], out_cnt.at[cid].at[pl.ds(r0, RPT)])


@functools.cache
def _sc_agg1():
    return functools.partial(
        pl.kernel,
        mesh=_mesh(),
        compiler_params=pltpu.CompilerParams(use_tc_tiling_on_sc=False),
        out_type=(jax.ShapeDtypeStruct((NC, NP, D), jnp.float32),
                  jax.ShapeDtypeStruct((NC, NP, CW), jnp.float32)),
        scratch_types=[
            pltpu.VMEM((NCHUNK, CH), jnp.int32),
            pltpu.VMEM((NCHUNK, CH), jnp.int32),
            pltpu.VMEM((CH, D), jnp.float32),
            pltpu.VMEM((CH, D), jnp.float32),
            pltpu.VMEM((CH, CW), jnp.float32),
            pltpu.VMEM_SHARED((NP, D), jnp.float32),
            pltpu.VMEM_SHARED((NP, CW), jnp.float32),
            pltpu.SemaphoreType.DMA,
            pltpu.SemaphoreType.DMA,
            pltpu.SemaphoreType.DMA,
            pltpu.SemaphoreType.DMA,
            pltpu.SemaphoreType.DMA,
            pltpu.SemaphoreType.DMA,
        ],
    )(functools.partial(_sc_agg_body, True))


@functools.cache
def _sc_agg2():
    return functools.partial(
        pl.kernel,
        mesh=_mesh(),
        compiler_params=pltpu.CompilerParams(use_tc_tiling_on_sc=False),
        out_type=jax.ShapeDtypeStruct((NC, NP, D), jnp.float32),
        scratch_types=[
            pltpu.VMEM((NCHUNK, CH), jnp.int32),
            pltpu.VMEM((NCHUNK, CH), jnp.int32),
            pltpu.VMEM((CH, D), jnp.float32),
            pltpu.VMEM((CH, D), jnp.float32),
            pltpu.VMEM_SHARED((NP, D), jnp.float32),
            pltpu.SemaphoreType.DMA,
            pltpu.SemaphoreType.DMA,
            pltpu.SemaphoreType.DMA,
            pltpu.SemaphoreType.DMA,
        ],
    )(functools.partial(_sc_agg_body, False))


def _proj_body(x_ref, wl_ref, wr_ref, p_ref, xr_ref):
    xv = x_ref[...]
    p_ref[...] = jnp.dot(xv, wl_ref[...], preferred_element_type=jnp.float32)
    xr_ref[...] = jnp.dot(xv, wr_ref[...], preferred_element_type=jnp.float32)


_tc_proj = pl.pallas_call(
    _proj_body,
    grid=(GRID,),
    in_specs=[
        pl.BlockSpec((RB, IN_D), lambda i: (i, 0)),
        pl.BlockSpec((IN_D, D), lambda i: (0, 0)),
        pl.BlockSpec((IN_D, D), lambda i: (0, 0)),
    ],
    out_specs=[
        pl.BlockSpec((RB, D), lambda i: (i, 0)),
        pl.BlockSpec((RB, D), lambda i: (i, 0)),
    ],
    out_shape=[jax.ShapeDtypeStruct((N, D), jnp.float32)] * 2,
)


def _mid_body(s_ref, c_ref, xr_ref, b_ref, wl2_ref, wr2_ref, p2_ref, hr2_ref):
    ssum = s_ref[0] + s_ref[1]
    cnt = c_ref[0, :, 0:1] + c_ref[1, :, 0:1]
    inv = 1.0 / jnp.maximum(cnt, 1.0)
    h = jnp.maximum(ssum * inv + b_ref[...] + xr_ref[...], 0.0)
    p2_ref[...] = jnp.dot(h, wl2_ref[...], preferred_element_type=jnp.float32)
    hr2_ref[...] = jnp.dot(h, wr2_ref[...], preferred_element_type=jnp.float32)


_tc_mid = pl.pallas_call(
    _mid_body,
    grid=(GRID,),
    in_specs=[
        pl.BlockSpec((NC, RB, D), lambda i: (0, i, 0)),
        pl.BlockSpec((NC, RB, CW), lambda i: (0, i, 0)),
        pl.BlockSpec((RB, D), lambda i: (i, 0)),
        pl.BlockSpec((1, D), lambda i: (0, 0)),
        pl.BlockSpec((D, D), lambda i: (0, 0)),
        pl.BlockSpec((D, D), lambda i: (0, 0)),
    ],
    out_specs=[
        pl.BlockSpec((RB, D), lambda i: (i, 0)),
        pl.BlockSpec((RB, D), lambda i: (i, 0)),
    ],
    out_shape=[jax.ShapeDtypeStruct((N, D), jnp.float32)] * 2,
)


def _out_body(s_ref, c_ref, hr_ref, b_ref, wc_ref, bc_ref, o_ref):
    ssum = s_ref[0] + s_ref[1]
    cnt = c_ref[0, :, 0:1] + c_ref[1, :, 0:1]
    inv = 1.0 / jnp.maximum(cnt, 1.0)
    h2 = jnp.maximum(ssum * inv + b_ref[...] + hr_ref[...], 0.0)
    o_ref[...] = jnp.dot(h2, wc_ref[...], preferred_element_type=jnp.float32) + bc_ref[...]


def _make_tc_out(out_dim):
    return pl.pallas_call(
        _out_body,
        grid=(GRID,),
        in_specs=[
            pl.BlockSpec((NC, RB, D), lambda i: (0, i, 0)),
            pl.BlockSpec((NC, RB, CW), lambda i: (0, i, 0)),
            pl.BlockSpec((RB, D), lambda i: (i, 0)),
            pl.BlockSpec((1, D), lambda i: (0, 0)),
            pl.BlockSpec((D, out_dim), lambda i: (0, 0)),
            pl.BlockSpec((1, out_dim), lambda i: (0, 0)),
        ],
        out_specs=pl.BlockSpec((RB, out_dim), lambda i: (i, 0)),
        out_shape=jax.ShapeDtypeStruct((N, out_dim), jnp.float32),
    )


def kernel(x, edge_index, W_l1, b_l1, W_r1, W_l2, b_l2, W_r2, W_c, b_c):
    src = edge_index[0].astype(jnp.int32).reshape(NW, NCHUNK, CH)
    dst = edge_index[1].astype(jnp.int32).reshape(NW, NCHUNK, CH)
    zrow = jnp.zeros((RPT, D), jnp.float32)
    zcnt = jnp.zeros((RPT, CW), jnp.float32)
    ones = jnp.ones((CH, CW), jnp.float32)

    p1, xr1 = _tc_proj(x, W_l1, W_r1)
    s1, c1 = _sc_agg1()(p1, src, dst, zrow, zcnt, ones)
    p2, hr2 = _tc_mid(s1, c1, xr1, b_l1.reshape(1, -1), W_l2, W_r2)
    s2 = _sc_agg2()(p2, src, dst, zrow)
    out = _make_tc_out(W_c.shape[1])(s2, c1, hr2, b_l2.reshape(1, -1),
                                     W_c, b_c.reshape(1, -1))
    return out


# speedup vs baseline: 14.3063x; 1.0487x over previous
import functools

import jax
import jax.numpy as jnp
from jax import lax
from jax.experimental import pallas as pl
from jax.experimental.pallas import tpu as pltpu
from jax.experimental.pallas import tpu_sc as plsc

N = 10000
E = 320000
IN_D = 128
D = 64
CW = 16

NC, NS = 2, 16
NW = NC * NS
EPW = E // NW
CH = 80
NCHUNK = EPW // CH
NP = 10240
RPT = NP // NS

RB = 2000
GRID = N // RB

@functools.cache
def _mesh():
    return plsc.VectorSubcoreMesh(core_axis_name="c", subcore_axis_name="s")


def _sc_agg_body(with_cnt, *refs):
    if with_cnt:
        (table, ei, zrow, zcnt, ones,
         out_sum, out_cnt,
         srcs_v, dsts_v, rows0, rows1, ones_v,
         acc_sh, cnt_sh,
         sg0, sg1, ss0, ss1, sc0, sc1) = refs
    else:
        (table, ei, zrow,
         out_sum,
         srcs_v, dsts_v, rows0, rows1,
         acc_sh,
         sg0, sg1, ss0, ss1) = refs
    cid = lax.axis_index("c")
    sid = lax.axis_index("s")
    wid = cid * NS + sid
    r0 = sid * RPT
    pltpu.sync_copy(zrow, acc_sh.at[pl.ds(r0, RPT)])
    if with_cnt:
        pltpu.sync_copy(zcnt, cnt_sh.at[pl.ds(r0, RPT)])
        pltpu.sync_copy(ones, ones_v)
    pltpu.sync_copy(ei.at[0, wid], srcs_v)
    pltpu.sync_copy(ei.at[1, wid], dsts_v)
    plsc.subcore_barrier()

    rows = (rows0, rows1)
    sg = (sg0, sg1)
    ss = (ss0, ss1)
    scs = (sc0, sc1) if with_cnt else None

    def wait_scatter(b, c):
        pltpu.make_async_copy(rows[b], acc_sh.at[dsts_v.at[c]], ss[b]).wait()
        if with_cnt:
            pltpu.make_async_copy(ones_v, cnt_sh.at[dsts_v.at[c]], scs[b]).wait()

    pltpu.async_copy(table.at[srcs_v.at[0]], rows0, sg0)

    def pair(g, carry):
        for b in (0, 1):
            c = g * 2 + b
            bo = 1 - b
            if b == 1:
                wait_scatter(bo, c)
            else:
                @pl.when(g > 0)
                def _():
                    wait_scatter(bo, c)
            pltpu.async_copy(table.at[srcs_v.at[c + 1]], rows[bo], sg[bo])
            if with_cnt:
                pltpu.async_copy(ones_v, cnt_sh.at[dsts_v.at[c]], scs[b], add=True)
            pltpu.make_async_copy(table.at[srcs_v.at[c]], rows[b], sg[b]).wait()
            pltpu.async_copy(rows[b], acc_sh.at[dsts_v.at[c]], ss[b], add=True)
        return carry

    lax.fori_loop(0, (NCHUNK - 1) // 2, pair, 0)
    cl = NCHUNK - 1
    wait_scatter(1, cl)
    pltpu.make_async_copy(table.at[srcs_v.at[cl]], rows0, sg0).wait()
    pltpu.sync_copy(rows0, acc_sh.at[dsts_v.at[cl]], add=True)
    if with_cnt:
        pltpu.sync_copy(ones_v, cnt_sh.at[dsts_v.at[cl]], add=True)
    plsc.subcore_barrier()
    pltpu.sync_copy(acc_sh.at[pl.ds(r0, RPT)], out_sum.at[cid].at[pl.ds(r0, RPT)])
    if with_cnt:
        pltpu.sync_copy(cnt_sh.at[pl.ds(r0, RPT)], out_cnt.at[cid].at[pl.ds(r0, RPT)])


@functools.cache
def _sc_agg1():
    return functools.partial(
        pl.kernel,
        mesh=_mesh(),
        compiler_params=pltpu.CompilerParams(use_tc_tiling_on_sc=False),
        out_type=(jax.ShapeDtypeStruct((NC, NP, D), jnp.float32),
                  jax.ShapeDtypeStruct((NC, NP, CW), jnp.float32)),
        scratch_types=[
            pltpu.VMEM((NCHUNK, CH), jnp.int32),
            pltpu.VMEM((NCHUNK, CH), jnp.int32),
            pltpu.VMEM((CH, D), jnp.float32),
            pltpu.VMEM((CH, D), jnp.float32),
            pltpu.VMEM((CH, CW), jnp.float32),
            pltpu.VMEM_SHARED((NP, D), jnp.float32),
            pltpu.VMEM_SHARED((NP, CW), jnp.float32),
            pltpu.SemaphoreType.DMA,
            pltpu.SemaphoreType.DMA,
            pltpu.SemaphoreType.DMA,
            pltpu.SemaphoreType.DMA,
            pltpu.SemaphoreType.DMA,
            pltpu.SemaphoreType.DMA,
        ],
    )(functools.partial(_sc_agg_body, True))


@functools.cache
def _sc_agg2():
    return functools.partial(
        pl.kernel,
        mesh=_mesh(),
        compiler_params=pltpu.CompilerParams(use_tc_tiling_on_sc=False),
        out_type=jax.ShapeDtypeStruct((NC, NP, D), jnp.float32),
        scratch_types=[
            pltpu.VMEM((NCHUNK, CH), jnp.int32),
            pltpu.VMEM((NCHUNK, CH), jnp.int32),
            pltpu.VMEM((CH, D), jnp.float32),
            pltpu.VMEM((CH, D), jnp.float32),
            pltpu.VMEM_SHARED((NP, D), jnp.float32),
            pltpu.SemaphoreType.DMA,
            pltpu.SemaphoreType.DMA,
            pltpu.SemaphoreType.DMA,
            pltpu.SemaphoreType.DMA,
        ],
    )(functools.partial(_sc_agg_body, False))


def _proj_body(x_ref, wl_ref, wr_ref, p_ref, xr_ref):
    xv = x_ref[...]
    p_ref[...] = jnp.dot(xv, wl_ref[...], preferred_element_type=jnp.float32)
    xr_ref[...] = jnp.dot(xv, wr_ref[...], preferred_element_type=jnp.float32)


_tc_proj = pl.pallas_call(
    _proj_body,
    grid=(GRID,),
    in_specs=[
        pl.BlockSpec((RB, IN_D), lambda i: (i, 0)),
        pl.BlockSpec((IN_D, D), lambda i: (0, 0)),
        pl.BlockSpec((IN_D, D), lambda i: (0, 0)),
    ],
    out_specs=[
        pl.BlockSpec((RB, D), lambda i: (i, 0)),
        pl.BlockSpec((RB, D), lambda i: (i, 0)),
    ],
    out_shape=[jax.ShapeDtypeStruct((N, D), jnp.float32)] * 2,
)


def _mid_body(s_ref, c_ref, xr_ref, b_ref, wl2_ref, wr2_ref, p2_ref, hr2_ref):
    ssum = s_ref[0] + s_ref[1]
    cnt = c_ref[0, :, 0:1] + c_ref[1, :, 0:1]
    inv = 1.0 / jnp.maximum(cnt, 1.0)
    h = jnp.maximum(ssum * inv + b_ref[...] + xr_ref[...], 0.0)
    p2_ref[...] = jnp.dot(h, wl2_ref[...], preferred_element_type=jnp.float32)
    hr2_ref[...] = jnp.dot(h, wr2_ref[...], preferred_element_type=jnp.float32)


_tc_mid = pl.pallas_call(
    _mid_body,
    grid=(GRID,),
    in_specs=[
        pl.BlockSpec((NC, RB, D), lambda i: (0, i, 0)),
        pl.BlockSpec((NC, RB, CW), lambda i: (0, i, 0)),
        pl.BlockSpec((RB, D), lambda i: (i, 0)),
        pl.BlockSpec((1, D), lambda i: (0, 0)),
        pl.BlockSpec((D, D), lambda i: (0, 0)),
        pl.BlockSpec((D, D), lambda i: (0, 0)),
    ],
    out_specs=[
        pl.BlockSpec((RB, D), lambda i: (i, 0)),
        pl.BlockSpec((RB, D), lambda i: (i, 0)),
    ],
    out_shape=[jax.ShapeDtypeStruct((N, D), jnp.float32)] * 2,
)


def _out_body(s_ref, c_ref, hr_ref, b_ref, wc_ref, bc_ref, o_ref):
    ssum = s_ref[0] + s_ref[1]
    cnt = c_ref[0, :, 0:1] + c_ref[1, :, 0:1]
    inv = 1.0 / jnp.maximum(cnt, 1.0)
    h2 = jnp.maximum(ssum * inv + b_ref[...] + hr_ref[...], 0.0)
    o_ref[...] = jnp.dot(h2, wc_ref[...], preferred_element_type=jnp.float32) + bc_ref[...]


def _make_tc_out(out_dim):
    return pl.pallas_call(
        _out_body,
        grid=(GRID,),
        in_specs=[
            pl.BlockSpec((NC, RB, D), lambda i: (0, i, 0)),
            pl.BlockSpec((NC, RB, CW), lambda i: (0, i, 0)),
            pl.BlockSpec((RB, D), lambda i: (i, 0)),
            pl.BlockSpec((1, D), lambda i: (0, 0)),
            pl.BlockSpec((D, out_dim), lambda i: (0, 0)),
            pl.BlockSpec((1, out_dim), lambda i: (0, 0)),
        ],
        out_specs=pl.BlockSpec((RB, out_dim), lambda i: (i, 0)),
        out_shape=jax.ShapeDtypeStruct((N, out_dim), jnp.float32),
    )


def kernel(x, edge_index, W_l1, b_l1, W_r1, W_l2, b_l2, W_r2, W_c, b_c):
    ei = edge_index.astype(jnp.int32).reshape(2, NW, NCHUNK, CH)
    zrow = jnp.zeros((RPT, D), jnp.float32)
    zcnt = jnp.zeros((RPT, CW), jnp.float32)
    ones = jnp.ones((CH, CW), jnp.float32)

    p1, xr1 = _tc_proj(x, W_l1, W_r1)
    s1, c1 = _sc_agg1()(p1, ei, zrow, zcnt, ones)
    p2, hr2 = _tc_mid(s1, c1, xr1, b_l1.reshape(1, -1), W_l2, W_r2)
    s2 = _sc_agg2()(p2, ei, zrow)
    out = _make_tc_out(W_c.shape[1])(s2, c1, hr2, b_l2.reshape(1, -1),
                                     W_c, b_c.reshape(1, -1))
    return out


# speedup vs baseline: 17.2916x; 1.2087x over previous
import functools

import jax
import jax.numpy as jnp
from jax import lax
from jax.experimental import pallas as pl
from jax.experimental.pallas import tpu as pltpu
from jax.experimental.pallas import tpu_sc as plsc

N = 10000
E = 320000
IN_D = 128
D = 64
CW = 16

NC, NS = 2, 16
NW = NC * NS
EPW = E // NW
CH = 80
NCHUNK = EPW // CH
NP = 10240
RD = 5
RPT = NP // NS

RB = 2000
GRID = N // RB

@functools.cache
def _mesh():
    return plsc.VectorSubcoreMesh(core_axis_name="c", subcore_axis_name="s")


def _sc_agg_body(with_cnt, *refs):
    if with_cnt:
        (table, ei, zrow, zcnt, ones,
         out_sum, out_cnt,
         srcs_v, dsts_v, ones_v) = refs[:10]
        rows = refs[10:10 + RD]
        acc_sh, cnt_sh = refs[10 + RD:12 + RD]
        sg = refs[12 + RD:12 + 2 * RD]
        ss = refs[12 + 2 * RD:12 + 3 * RD]
        scs = refs[12 + 3 * RD:12 + 4 * RD]
    else:
        (table, ei, zrow,
         out_sum,
         srcs_v, dsts_v) = refs[:6]
        rows = refs[6:6 + RD]
        acc_sh = refs[6 + RD]
        sg = refs[7 + RD:7 + 2 * RD]
        ss = refs[7 + 2 * RD:7 + 3 * RD]
        scs = None
    cid = lax.axis_index("c")
    sid = lax.axis_index("s")
    wid = cid * NS + sid
    r0 = sid * RPT
    pltpu.sync_copy(zrow, acc_sh.at[pl.ds(r0, RPT)])
    if with_cnt:
        pltpu.sync_copy(zcnt, cnt_sh.at[pl.ds(r0, RPT)])
        pltpu.sync_copy(ones, ones_v)
    pltpu.sync_copy(ei.at[0, wid], srcs_v)
    pltpu.sync_copy(ei.at[1, wid], dsts_v)
    plsc.subcore_barrier()

    def wait_scatter(b, c):
        pltpu.make_async_copy(rows[b], acc_sh.at[dsts_v.at[c]], ss[b]).wait()
        if with_cnt:
            pltpu.make_async_copy(ones_v, cnt_sh.at[dsts_v.at[c]], scs[b]).wait()

    pltpu.async_copy(table.at[srcs_v.at[0]], rows[0], sg[0])
    pltpu.async_copy(table.at[srcs_v.at[1]], rows[1], sg[1])

    def ring(g, carry):
        for b in range(RD):
            c = g * RD + b

            def step(c=c, b=b):
                wait_scatter((b + 2) % RD, c)
                pltpu.async_copy(table.at[srcs_v.at[c + 2]],
                                 rows[(b + 2) % RD], sg[(b + 2) % RD])

            if b < 3:
                @pl.when(g >= 1)
                def _():
                    step()
                @pl.when(g == 0)
                def _():
                    pltpu.async_copy(table.at[srcs_v.at[c + 2]],
                                     rows[(b + 2) % RD], sg[(b + 2) % RD])
            else:
                @pl.when(g < (NCHUNK // RD) - 1)
                def _():
                    step()
                @pl.when(g == (NCHUNK // RD) - 1)
                def _():
                    wait_scatter((b + 2) % RD, c)
            if with_cnt:
                pltpu.async_copy(ones_v, cnt_sh.at[dsts_v.at[c]], scs[b], add=True)
            pltpu.make_async_copy(table.at[srcs_v.at[c]], rows[b], sg[b]).wait()
            pltpu.async_copy(rows[b], acc_sh.at[dsts_v.at[c]], ss[b], add=True)
        return carry

    lax.fori_loop(0, NCHUNK // RD, ring, 0)
    for c in (NCHUNK - 3, NCHUNK - 2, NCHUNK - 1):
        wait_scatter(c % RD, c)
    plsc.subcore_barrier()
    pltpu.sync_copy(acc_sh.at[pl.ds(r0, RPT)], out_sum.at[cid].at[pl.ds(r0, RPT)])
    if with_cnt:
        pltpu.sync_copy(cnt_sh.at[pl.ds(r0, RPT)], out_cnt.at[cid].at[pl.ds(r0, RPT)])


@functools.cache
def _sc_agg1():
    return functools.partial(
        pl.kernel,
        mesh=_mesh(),
        compiler_params=pltpu.CompilerParams(use_tc_tiling_on_sc=False),
        out_type=(jax.ShapeDtypeStruct((NC, NP, D), jnp.float32),
                  jax.ShapeDtypeStruct((NC, NP, CW), jnp.float32)),
        scratch_types=(
            [pltpu.VMEM((NCHUNK, CH), jnp.int32),
             pltpu.VMEM((NCHUNK, CH), jnp.int32),
             pltpu.VMEM((CH, CW), jnp.float32)]
            + [pltpu.VMEM((CH, D), jnp.float32)] * RD
            + [pltpu.VMEM_SHARED((NP, D), jnp.float32),
               pltpu.VMEM_SHARED((NP, CW), jnp.float32)]
            + [pltpu.SemaphoreType.DMA] * (3 * RD)
        ),
    )(functools.partial(_sc_agg_body, True))


@functools.cache
def _sc_agg2():
    return functools.partial(
        pl.kernel,
        mesh=_mesh(),
        compiler_params=pltpu.CompilerParams(use_tc_tiling_on_sc=False),
        out_type=jax.ShapeDtypeStruct((NC, NP, D), jnp.float32),
        scratch_types=(
            [pltpu.VMEM((NCHUNK, CH), jnp.int32),
             pltpu.VMEM((NCHUNK, CH), jnp.int32)]
            + [pltpu.VMEM((CH, D), jnp.float32)] * RD
            + [pltpu.VMEM_SHARED((NP, D), jnp.float32)]
            + [pltpu.SemaphoreType.DMA] * (2 * RD)
        ),
    )(functools.partial(_sc_agg_body, False))


def _proj_body(x_ref, wl_ref, wr_ref, p_ref, xr_ref):
    xv = x_ref[...]
    p_ref[...] = jnp.dot(xv, wl_ref[...], preferred_element_type=jnp.float32)
    xr_ref[...] = jnp.dot(xv, wr_ref[...], preferred_element_type=jnp.float32)


_tc_proj = pl.pallas_call(
    _proj_body,
    grid=(GRID,),
    in_specs=[
        pl.BlockSpec((RB, IN_D), lambda i: (i, 0)),
        pl.BlockSpec((IN_D, D), lambda i: (0, 0)),
        pl.BlockSpec((IN_D, D), lambda i: (0, 0)),
    ],
    out_specs=[
        pl.BlockSpec((RB, D), lambda i: (i, 0)),
        pl.BlockSpec((RB, D), lambda i: (i, 0)),
    ],
    out_shape=[jax.ShapeDtypeStruct((N, D), jnp.float32)] * 2,
)


def _mid_body(s_ref, c_ref, xr_ref, b_ref, wl2_ref, wr2_ref, p2_ref, hr2_ref):
    ssum = s_ref[0] + s_ref[1]
    cnt = c_ref[0, :, 0:1] + c_ref[1, :, 0:1]
    inv = 1.0 / jnp.maximum(cnt, 1.0)
    h = jnp.maximum(ssum * inv + b_ref[...] + xr_ref[...], 0.0)
    p2_ref[...] = jnp.dot(h, wl2_ref[...], preferred_element_type=jnp.float32)
    hr2_ref[...] = jnp.dot(h, wr2_ref[...], preferred_element_type=jnp.float32)


_tc_mid = pl.pallas_call(
    _mid_body,
    grid=(GRID,),
    in_specs=[
        pl.BlockSpec((NC, RB, D), lambda i: (0, i, 0)),
        pl.BlockSpec((NC, RB, CW), lambda i: (0, i, 0)),
        pl.BlockSpec((RB, D), lambda i: (i, 0)),
        pl.BlockSpec((1, D), lambda i: (0, 0)),
        pl.BlockSpec((D, D), lambda i: (0, 0)),
        pl.BlockSpec((D, D), lambda i: (0, 0)),
    ],
    out_specs=[
        pl.BlockSpec((RB, D), lambda i: (i, 0)),
        pl.BlockSpec((RB, D), lambda i: (i, 0)),
    ],
    out_shape=[jax.ShapeDtypeStruct((N, D), jnp.float32)] * 2,
)


def _out_body(s_ref, c_ref, hr_ref, b_ref, wc_ref, bc_ref, o_ref):
    ssum = s_ref[0] + s_ref[1]
    cnt = c_ref[0, :, 0:1] + c_ref[1, :, 0:1]
    inv = 1.0 / jnp.maximum(cnt, 1.0)
    h2 = jnp.maximum(ssum * inv + b_ref[...] + hr_ref[...], 0.0)
    o_ref[...] = jnp.dot(h2, wc_ref[...], preferred_element_type=jnp.float32) + bc_ref[...]


def _make_tc_out(out_dim):
    return pl.pallas_call(
        _out_body,
        grid=(GRID,),
        in_specs=[
            pl.BlockSpec((NC, RB, D), lambda i: (0, i, 0)),
            pl.BlockSpec((NC, RB, CW), lambda i: (0, i, 0)),
            pl.BlockSpec((RB, D), lambda i: (i, 0)),
            pl.BlockSpec((1, D), lambda i: (0, 0)),
            pl.BlockSpec((D, out_dim), lambda i: (0, 0)),
            pl.BlockSpec((1, out_dim), lambda i: (0, 0)),
        ],
        out_specs=pl.BlockSpec((RB, out_dim), lambda i: (i, 0)),
        out_shape=jax.ShapeDtypeStruct((N, out_dim), jnp.float32),
    )


def kernel(x, edge_index, W_l1, b_l1, W_r1, W_l2, b_l2, W_r2, W_c, b_c):
    ei = edge_index.astype(jnp.int32).reshape(2, NW, NCHUNK, CH)
    zrow = jnp.zeros((RPT, D), jnp.float32)
    zcnt = jnp.zeros((RPT, CW), jnp.float32)
    ones = jnp.ones((CH, CW), jnp.float32)

    p1, xr1 = _tc_proj(x, W_l1, W_r1)
    s1, c1 = _sc_agg1()(p1, ei, zrow, zcnt, ones)
    p2, hr2 = _tc_mid(s1, c1, xr1, b_l1.reshape(1, -1), W_l2, W_r2)
    s2 = _sc_agg2()(p2, ei, zrow)
    out = _make_tc_out(W_c.shape[1])(s2, c1, hr2, b_l2.reshape(1, -1),
                                     W_c, b_c.reshape(1, -1))
    return out


# speedup vs baseline: 18.6784x; 1.0802x over previous
import functools

import jax
import jax.numpy as jnp
from jax import lax
from jax.experimental import pallas as pl
from jax.experimental.pallas import tpu as pltpu
from jax.experimental.pallas import tpu_sc as plsc

N = 10000
E = 320000
IN_D = 128
D = 64
CW = 16

NC, NS = 2, 16
NW = NC * NS
EPW = E // NW
CH = 80
NCHUNK = EPW // CH
NP = 10240
RD = 5
RPT = NP // NS

RB = 2048
PB = RB // 2
GRID = NP // RB
NPK = NP // 2

@functools.cache
def _mesh():
    return plsc.VectorSubcoreMesh(core_axis_name="c", subcore_axis_name="s")


def _sc_agg_body(with_cnt, *refs):
    if with_cnt:
        (table, ei, zrow, zcnt, ones,
         out_sum, out_cnt,
         srcs_v, dsts_v, ones_v) = refs[:10]
        rows = refs[10:10 + RD]
        acc_sh, cnt_sh = refs[10 + RD:12 + RD]
        sg = refs[12 + RD:12 + 2 * RD]
        ss = refs[12 + 2 * RD:12 + 3 * RD]
        scs = refs[12 + 3 * RD:12 + 4 * RD]
    else:
        (table, ei, zrow,
         out_sum,
         srcs_v, dsts_v) = refs[:6]
        rows = refs[6:6 + RD]
        acc_sh = refs[6 + RD]
        sg = refs[7 + RD:7 + 2 * RD]
        ss = refs[7 + 2 * RD:7 + 3 * RD]
        scs = None
    cid = lax.axis_index("c")
    sid = lax.axis_index("s")
    wid = cid * NS + sid
    r0 = sid * RPT
    pltpu.sync_copy(zrow, acc_sh.at[pl.ds(r0, RPT)])
    if with_cnt:
        pltpu.sync_copy(zcnt, cnt_sh.at[pl.ds(r0, RPT)])
        pltpu.sync_copy(ones, ones_v)
    pltpu.sync_copy(ei.at[0, wid], srcs_v)
    pltpu.sync_copy(ei.at[1, wid], dsts_v)
    plsc.subcore_barrier()

    def wait_scatter(b, c):
        pltpu.make_async_copy(rows[b], acc_sh.at[dsts_v.at[c]], ss[b]).wait()
        if with_cnt:
            pltpu.make_async_copy(ones_v, cnt_sh.at[dsts_v.at[c]], scs[b]).wait()

    pltpu.async_copy(table.at[srcs_v.at[0]], rows[0], sg[0])
    pltpu.async_copy(table.at[srcs_v.at[1]], rows[1], sg[1])

    def ring(g, carry):
        for b in range(RD):
            c = g * RD + b

            def step(c=c, b=b):
                wait_scatter((b + 2) % RD, c)
                pltpu.async_copy(table.at[srcs_v.at[c + 2]],
                                 rows[(b + 2) % RD], sg[(b + 2) % RD])

            if b < 3:
                @pl.when(g >= 1)
                def _():
                    step()
                @pl.when(g == 0)
                def _():
                    pltpu.async_copy(table.at[srcs_v.at[c + 2]],
                                     rows[(b + 2) % RD], sg[(b + 2) % RD])
            else:
                @pl.when(g < (NCHUNK // RD) - 1)
                def _():
                    step()
                @pl.when(g == (NCHUNK // RD) - 1)
                def _():
                    wait_scatter((b + 2) % RD, c)
            if with_cnt:
                pltpu.async_copy(ones_v, cnt_sh.at[dsts_v.at[c]], scs[b], add=True)
            pltpu.make_async_copy(table.at[srcs_v.at[c]], rows[b], sg[b]).wait()
            pltpu.async_copy(rows[b], acc_sh.at[dsts_v.at[c]], ss[b], add=True)
        return carry

    lax.fori_loop(0, NCHUNK // RD, ring, 0)
    for c in (NCHUNK - 3, NCHUNK - 2, NCHUNK - 1):
        wait_scatter(c % RD, c)
    plsc.subcore_barrier()
    pltpu.sync_copy(acc_sh.at[pl.ds(r0, RPT)], out_sum.at[cid].at[pl.ds(r0, RPT)])
    if with_cnt:
        pltpu.sync_copy(cnt_sh.at[pl.ds(r0, RPT)], out_cnt.at[cid].at[pl.ds(r0, RPT)])


@functools.cache
def _sc_agg1():
    return functools.partial(
        pl.kernel,
        mesh=_mesh(),
        compiler_params=pltpu.CompilerParams(use_tc_tiling_on_sc=False),
        out_type=(jax.ShapeDtypeStruct((NC, NP, D), jnp.float32),
                  jax.ShapeDtypeStruct((NC, NP, CW), jnp.float32)),
        scratch_types=(
            [pltpu.VMEM((NCHUNK, CH), jnp.int32),
             pltpu.VMEM((NCHUNK, CH), jnp.int32),
             pltpu.VMEM((CH, CW), jnp.float32)]
            + [pltpu.VMEM((CH, D), jnp.float32)] * RD
            + [pltpu.VMEM_SHARED((NP, D), jnp.float32),
               pltpu.VMEM_SHARED((NP, CW), jnp.float32)]
            + [pltpu.SemaphoreType.DMA] * (3 * RD)
        ),
    )(functools.partial(_sc_agg_body, True))


@functools.cache
def _sc_agg2():
    return functools.partial(
        pl.kernel,
        mesh=_mesh(),
        compiler_params=pltpu.CompilerParams(use_tc_tiling_on_sc=False),
        out_type=jax.ShapeDtypeStruct((NC, NP, D), jnp.float32),
        scratch_types=(
            [pltpu.VMEM((NCHUNK, CH), jnp.int32),
             pltpu.VMEM((NCHUNK, CH), jnp.int32)]
            + [pltpu.VMEM((CH, D), jnp.float32)] * RD
            + [pltpu.VMEM_SHARED((NP, D), jnp.float32)]
            + [pltpu.SemaphoreType.DMA] * (2 * RD)
        ),
    )(functools.partial(_sc_agg_body, False))


def _eo(v, n, w):
    v3 = v.reshape(n // 2, 2, w)
    return v3[:, 0, :], v3[:, 1, :]


def _pair_dot(ve, vo, w):
    return jnp.concatenate(
        [jnp.dot(ve, w, preferred_element_type=jnp.float32),
         jnp.dot(vo, w, preferred_element_type=jnp.float32)], axis=1)


def _proj_body(x_ref, wl_ref, wr_ref, p_ref, xr_ref):
    xv = x_ref[...]
    xe, xo = _eo(xv, RB, IN_D)
    p_ref[...] = _pair_dot(xe, xo, wl_ref[...])
    xr_ref[...] = _pair_dot(xe, xo, wr_ref[...])


_tc_proj = pl.pallas_call(
    _proj_body,
    grid=(GRID,),
    in_specs=[
        pl.BlockSpec((RB, IN_D), lambda i: (i, 0)),
        pl.BlockSpec((IN_D, D), lambda i: (0, 0)),
        pl.BlockSpec((IN_D, D), lambda i: (0, 0)),
    ],
    out_specs=[
        pl.BlockSpec((PB, 2 * D), lambda i: (i, 0)),
        pl.BlockSpec((PB, 2 * D), lambda i: (i, 0)),
    ],
    out_shape=[jax.ShapeDtypeStruct((NPK, 2 * D), jnp.float32)] * 2,
)


def _packed_inv(c_ref):
    cnt = (c_ref[0] + c_ref[1])[:, 0:1]
    inv = 1.0 / jnp.maximum(cnt, 1.0)
    ie, io = _eo(inv, RB, 1)
    return jnp.concatenate([jnp.broadcast_to(ie, (PB, D)),
                            jnp.broadcast_to(io, (PB, D))], axis=1)


def _mid_body(s_ref, c_ref, xr_ref, b_ref, wl2_ref, wr2_ref, p2_ref, hr2_ref):
    ssum = s_ref[0] + s_ref[1]
    h = jnp.maximum(ssum * _packed_inv(c_ref) + b_ref[...] + xr_ref[...], 0.0)
    he = h[:, :D]
    ho = h[:, D:]
    p2_ref[...] = _pair_dot(he, ho, wl2_ref[...])
    hr2_ref[...] = _pair_dot(he, ho, wr2_ref[...])


_tc_mid = pl.pallas_call(
    _mid_body,
    grid=(GRID,),
    in_specs=[
        pl.BlockSpec((NC, PB, 2 * D), lambda i: (0, i, 0)),
        pl.BlockSpec((NC, RB, CW), lambda i: (0, i, 0)),
        pl.BlockSpec((PB, 2 * D), lambda i: (i, 0)),
        pl.BlockSpec((1, 2 * D), lambda i: (0, 0)),
        pl.BlockSpec((D, D), lambda i: (0, 0)),
        pl.BlockSpec((D, D), lambda i: (0, 0)),
    ],
    out_specs=[
        pl.BlockSpec((PB, 2 * D), lambda i: (i, 0)),
        pl.BlockSpec((PB, 2 * D), lambda i: (i, 0)),
    ],
    out_shape=[jax.ShapeDtypeStruct((NPK, 2 * D), jnp.float32)] * 2,
)


def _out_body(s_ref, c_ref, hr_ref, b_ref, wc_ref, bc_ref, o_ref):
    ssum = s_ref[0] + s_ref[1]
    h2 = jnp.maximum(ssum * _packed_inv(c_ref) + b_ref[...] + hr_ref[...], 0.0)
    he = h2[:, :D]
    ho = h2[:, D:]
    o_ref[...] = _pair_dot(he, ho, wc_ref[...]) + bc_ref[...]


def _make_tc_out(out_dim):
    return pl.pallas_call(
        _out_body,
        grid=(GRID,),
        in_specs=[
            pl.BlockSpec((NC, PB, 2 * D), lambda i: (0, i, 0)),
            pl.BlockSpec((NC, RB, CW), lambda i: (0, i, 0)),
            pl.BlockSpec((PB, 2 * D), lambda i: (i, 0)),
            pl.BlockSpec((1, 2 * D), lambda i: (0, 0)),
            pl.BlockSpec((D, out_dim), lambda i: (0, 0)),
            pl.BlockSpec((1, 2 * out_dim), lambda i: (0, 0)),
        ],
        out_specs=pl.BlockSpec((PB, 2 * out_dim), lambda i: (i, 0)),
        out_shape=jax.ShapeDtypeStruct((NPK, 2 * out_dim), jnp.float32),
    )


def kernel(x, edge_index, W_l1, b_l1, W_r1, W_l2, b_l2, W_r2, W_c, b_c):
    ei = edge_index.astype(jnp.int32).reshape(2, NW, NCHUNK, CH)
    zrow = jnp.zeros((RPT, D), jnp.float32)
    zcnt = jnp.zeros((RPT, CW), jnp.float32)
    ones = jnp.ones((CH, CW), jnp.float32)

    b1p = jnp.concatenate([b_l1, b_l1]).reshape(1, 2 * D)
    b2p = jnp.concatenate([b_l2, b_l2]).reshape(1, 2 * D)
    bcp = jnp.concatenate([b_c, b_c]).reshape(1, -1)

    p1p, xr1p = _tc_proj(x, W_l1, W_r1)
    s1, c1 = _sc_agg1()(p1p.reshape(NP, D), ei, zrow, zcnt, ones)
    s1p = s1.reshape(NC, NPK, 2 * D)
    p2p, hr2p = _tc_mid(s1p, c1, xr1p, b1p, W_l2, W_r2)
    s2 = _sc_agg2()(p2p.reshape(NP, D), ei, zrow)
    s2p = s2.reshape(NC, NPK, 2 * D)
    outp = _make_tc_out(W_c.shape[1])(s2p, c1, hr2p, b2p, W_c, bcp)
    return outp.reshape(NP, -1)[:N]


# speedup vs baseline: 19.2273x; 1.0294x over previous
import functools

import numpy as np

import jax
import jax.numpy as jnp
from jax import lax
from jax.experimental import pallas as pl
from jax.experimental.pallas import tpu as pltpu
from jax.experimental.pallas import tpu_sc as plsc

N = 10000
E = 320000
IN_D = 128
D = 64
CW = 16

NC, NS = 2, 16
NW = NC * NS
EPW = E // NW
CH = 80
NCHUNK = EPW // CH
NP = 10240
RD = 5
RPT = NP // NS

RB = 2048
PB = RB // 2
GRID = NP // RB
NPK = NP // 2

_ZROW = np.zeros((RPT, D), np.float32)
_ZCNT = np.zeros((RPT, CW), np.float32)
_ONES = np.ones((CH, CW), np.float32)


@functools.cache
def _mesh():
    return plsc.VectorSubcoreMesh(core_axis_name="c", subcore_axis_name="s")


def _sc_agg_body(with_cnt, *refs):
    if with_cnt:
        (table, ei, zrow, zcnt, ones,
         out_sum, out_cnt,
         srcs_v, dsts_v, ones_v) = refs[:10]
        rows = refs[10:10 + RD]
        acc_sh, cnt_sh = refs[10 + RD:12 + RD]
        sg = refs[12 + RD:12 + 2 * RD]
        ss = refs[12 + 2 * RD:12 + 3 * RD]
        scs = refs[12 + 3 * RD:12 + 4 * RD]
    else:
        (table, ei, zrow,
         out_sum,
         srcs_v, dsts_v) = refs[:6]
        rows = refs[6:6 + RD]
        acc_sh = refs[6 + RD]
        sg = refs[7 + RD:7 + 2 * RD]
        ss = refs[7 + 2 * RD:7 + 3 * RD]
        scs = None
    cid = lax.axis_index("c")
    sid = lax.axis_index("s")
    wid = cid * NS + sid
    r0 = sid * RPT
    pltpu.sync_copy(zrow, acc_sh.at[pl.ds(r0, RPT)])
    if with_cnt:
        pltpu.sync_copy(zcnt, cnt_sh.at[pl.ds(r0, RPT)])
        pltpu.sync_copy(ones, ones_v)
    pltpu.sync_copy(ei.at[0, wid], srcs_v)
    pltpu.sync_copy(ei.at[1, wid], dsts_v)
    plsc.subcore_barrier()

    def wait_scatter(b, c):
        pltpu.make_async_copy(rows[b], acc_sh.at[dsts_v.at[c]], ss[b]).wait()
        if with_cnt:
            pltpu.make_async_copy(ones_v, cnt_sh.at[dsts_v.at[c]], scs[b]).wait()

    pltpu.async_copy(table.at[srcs_v.at[0]], rows[0], sg[0])
    pltpu.async_copy(table.at[srcs_v.at[1]], rows[1], sg[1])
    pltpu.async_copy(table.at[srcs_v.at[2]], rows[2], sg[2])

    def ring(g, carry):
        for b in range(RD):
            c = g * RD + b

            def step(c=c, b=b):
                wait_scatter((b + 3) % RD, c)
                pltpu.async_copy(table.at[srcs_v.at[c + 3]],
                                 rows[(b + 3) % RD], sg[(b + 3) % RD])

            if b < 2:
                @pl.when(g >= 1)
                def _():
                    step()
                @pl.when(g == 0)
                def _():
                    pltpu.async_copy(table.at[srcs_v.at[c + 3]],
                                     rows[(b + 3) % RD], sg[(b + 3) % RD])
            else:
                @pl.when(g < (NCHUNK // RD) - 1)
                def _():
                    step()
                @pl.when(g == (NCHUNK // RD) - 1)
                def _():
                    wait_scatter((b + 3) % RD, c)
            if with_cnt:
                pltpu.async_copy(ones_v, cnt_sh.at[dsts_v.at[c]], scs[b], add=True)
            pltpu.make_async_copy(table.at[srcs_v.at[c]], rows[b], sg[b]).wait()
            pltpu.async_copy(rows[b], acc_sh.at[dsts_v.at[c]], ss[b], add=True)
        return carry

    lax.fori_loop(0, NCHUNK // RD, ring, 0)
    for c in (NCHUNK - 2, NCHUNK - 1):
        wait_scatter(c % RD, c)
    plsc.subcore_barrier()
    pltpu.sync_copy(acc_sh.at[pl.ds(r0, RPT)], out_sum.at[cid].at[pl.ds(r0, RPT)])
    if with_cnt:
        pltpu.sync_copy(cnt_sh.at[pl.ds(r0, RPT)], out_cnt.at[cid].at[pl.ds(r0, RPT)])


@functools.cache
def _sc_agg1():
    return functools.partial(
        pl.kernel,
        mesh=_mesh(),
        compiler_params=pltpu.CompilerParams(use_tc_tiling_on_sc=False),
        out_type=(jax.ShapeDtypeStruct((NC, NP, D), jnp.float32),
                  jax.ShapeDtypeStruct((NC, NP, CW), jnp.float32)),
        scratch_types=(
            [pltpu.VMEM((NCHUNK, CH), jnp.int32),
             pltpu.VMEM((NCHUNK, CH), jnp.int32),
             pltpu.VMEM((CH, CW), jnp.float32)]
            + [pltpu.VMEM((CH, D), jnp.float32)] * RD
            + [pltpu.VMEM_SHARED((NP, D), jnp.float32),
               pltpu.VMEM_SHARED((NP, CW), jnp.float32)]
            + [pltpu.SemaphoreType.DMA] * (3 * RD)
        ),
    )(functools.partial(_sc_agg_body, True))


@functools.cache
def _sc_agg2():
    return functools.partial(
        pl.kernel,
        mesh=_mesh(),
        compiler_params=pltpu.CompilerParams(use_tc_tiling_on_sc=False),
        out_type=jax.ShapeDtypeStruct((NC, NP, D), jnp.float32),
        scratch_types=(
            [pltpu.VMEM((NCHUNK, CH), jnp.int32),
             pltpu.VMEM((NCHUNK, CH), jnp.int32)]
            + [pltpu.VMEM((CH, D), jnp.float32)] * RD
            + [pltpu.VMEM_SHARED((NP, D), jnp.float32)]
            + [pltpu.SemaphoreType.DMA] * (2 * RD)
        ),
    )(functools.partial(_sc_agg_body, False))


def _eo(v, n, w):
    v3 = v.reshape(n // 2, 2, w)
    return v3[:, 0, :], v3[:, 1, :]


def _pair_dot(ve, vo, w):
    return jnp.concatenate(
        [jnp.dot(ve, w, preferred_element_type=jnp.float32),
         jnp.dot(vo, w, preferred_element_type=jnp.float32)], axis=1)


def _proj_body(x_ref, w_ref, p_ref):
    xe, xo = _eo(x_ref[...], RB, IN_D)
    p_ref[...] = _pair_dot(xe, xo, w_ref[...])


_tc_proj = pl.pallas_call(
    _proj_body,
    grid=(GRID,),
    in_specs=[
        pl.BlockSpec((RB, IN_D), lambda i: (i, 0)),
        pl.BlockSpec((IN_D, D), lambda i: (0, 0)),
    ],
    out_specs=pl.BlockSpec((PB, 2 * D), lambda i: (i, 0)),
    out_shape=jax.ShapeDtypeStruct((NPK, 2 * D), jnp.float32),
)


def _packed_inv(c_ref):
    cnt = (c_ref[0] + c_ref[1])[:, 0:1]
    inv = 1.0 / jnp.maximum(cnt, 1.0)
    ie, io = _eo(inv, RB, 1)
    return jnp.concatenate([jnp.broadcast_to(ie, (PB, D)),
                            jnp.broadcast_to(io, (PB, D))], axis=1)


def _mid_body(s_ref, c_ref, xr_ref, b_ref, w_ref, p2_ref):
    ssum = s_ref[0] + s_ref[1]
    h = jnp.maximum(ssum * _packed_inv(c_ref) + b_ref[...] + xr_ref[...], 0.0)
    p2_ref[...] = _pair_dot(h[:, :D], h[:, D:], w_ref[...])


_tc_mid = pl.pallas_call(
    _mid_body,
    grid=(GRID,),
    in_specs=[
        pl.BlockSpec((NC, PB, 2 * D), lambda i: (0, i, 0)),
        pl.BlockSpec((NC, RB, CW), lambda i: (0, i, 0)),
        pl.BlockSpec((PB, 2 * D), lambda i: (i, 0)),
        pl.BlockSpec((1, 2 * D), lambda i: (0, 0)),
        pl.BlockSpec((D, D), lambda i: (0, 0)),
    ],
    out_specs=pl.BlockSpec((PB, 2 * D), lambda i: (i, 0)),
    out_shape=jax.ShapeDtypeStruct((NPK, 2 * D), jnp.float32),
)


def _out_body(s_ref, c_ref, hr_ref, b_ref, wc_ref, bc_ref, o_ref):
    ssum = s_ref[0] + s_ref[1]
    h2 = jnp.maximum(ssum * _packed_inv(c_ref) + b_ref[...] + hr_ref[...], 0.0)
    he = h2[:, :D]
    ho = h2[:, D:]
    o_ref[...] = _pair_dot(he, ho, wc_ref[...]) + bc_ref[...]


def _make_tc_out(out_dim):
    return pl.pallas_call(
        _out_body,
        grid=(GRID,),
        in_specs=[
            pl.BlockSpec((NC, PB, 2 * D), lambda i: (0, i, 0)),
            pl.BlockSpec((NC, RB, CW), lambda i: (0, i, 0)),
            pl.BlockSpec((PB, 2 * D), lambda i: (i, 0)),
            pl.BlockSpec((1, 2 * D), lambda i: (0, 0)),
            pl.BlockSpec((D, out_dim), lambda i: (0, 0)),
            pl.BlockSpec((1, 2 * out_dim), lambda i: (0, 0)),
        ],
        out_specs=pl.BlockSpec((PB, 2 * out_dim), lambda i: (i, 0)),
        out_shape=jax.ShapeDtypeStruct((NPK, 2 * out_dim), jnp.float32),
    )


def kernel(x, edge_index, W_l1, b_l1, W_r1, W_l2, b_l2, W_r2, W_c, b_c):
    if edge_index.dtype != jnp.int32:
        edge_index = edge_index.astype(jnp.int32)
    ei = edge_index.reshape(2, NW, NCHUNK, CH)

    b1p = jnp.concatenate([b_l1, b_l1]).reshape(1, 2 * D)
    b2p = jnp.concatenate([b_l2, b_l2]).reshape(1, 2 * D)
    bcp = jnp.concatenate([b_c, b_c]).reshape(1, -1)

    p1p = _tc_proj(x, W_l1)
    xr1p = _tc_proj(x, W_r1)
    s1, c1 = _sc_agg1()(p1p.reshape(NP, D), ei, _ZROW, _ZCNT, _ONES)
    s1p = s1.reshape(NC, NPK, 2 * D)
    p2p = _tc_mid(s1p, c1, xr1p, b1p, W_l2)
    hr2p = _tc_mid(s1p, c1, xr1p, b1p, W_r2)
    s2 = _sc_agg2()(p2p.reshape(NP, D), ei, _ZROW)
    s2p = s2.reshape(NC, NPK, 2 * D)
    outp = _make_tc_out(W_c.shape[1])(s2p, c1, hr2p, b2p, W_c, bcp)
    return outp.reshape(NP, -1)[:N]


# speedup vs baseline: 19.3429x; 1.0060x over previous
import functools

import numpy as np

import jax
import jax.numpy as jnp
from jax import lax
from jax.experimental import pallas as pl
from jax.experimental.pallas import tpu as pltpu
from jax.experimental.pallas import tpu_sc as plsc

N = 10000
E = 320000
IN_D = 128
D = 64
CW = 16

NC, NS = 2, 16
NW = NC * NS
EPW = E // NW
CH = 80
NCHUNK = EPW // CH
NP = 10240
RD = 5
RPT = NP // NS

RB = 2048
PB = RB // 2
GRID = NP // RB
NPK = NP // 2

_ZROW = np.zeros((RPT, D), np.float32)
_ZCNT = np.zeros((RPT, CW), np.float32)
_ONES = np.ones((CH, CW), np.float32)


@functools.cache
def _mesh():
    return plsc.VectorSubcoreMesh(core_axis_name="c", subcore_axis_name="s")


def _sc_agg_body(with_cnt, *refs):
    if with_cnt:
        (table, ei, zrow, zcnt, ones,
         out_sum, out_cnt,
         srcs_v, dsts_v, ones_v) = refs[:10]
        rows = refs[10:10 + RD]
        acc_sh, cnt_sh = refs[10 + RD:12 + RD]
        sg = refs[12 + RD:12 + 2 * RD]
        ss = refs[12 + 2 * RD:12 + 3 * RD]
        scs = refs[12 + 3 * RD:12 + 4 * RD]
    else:
        (table, ei, zrow,
         out_sum,
         srcs_v, dsts_v) = refs[:6]
        rows = refs[6:6 + RD]
        acc_sh = refs[6 + RD]
        sg = refs[7 + RD:7 + 2 * RD]
        ss = refs[7 + 2 * RD:7 + 3 * RD]
        scs = None
    cid = lax.axis_index("c")
    sid = lax.axis_index("s")
    wid = cid * NS + sid
    r0 = sid * RPT
    pltpu.sync_copy(zrow, acc_sh.at[pl.ds(r0, RPT)])
    if with_cnt:
        pltpu.sync_copy(zcnt, cnt_sh.at[pl.ds(r0, RPT)])
        pltpu.sync_copy(ones, ones_v)
    pltpu.sync_copy(ei.at[0, wid], srcs_v)
    pltpu.sync_copy(ei.at[1, wid], dsts_v)
    plsc.subcore_barrier()

    def wait_scatter(b, c):
        pltpu.make_async_copy(rows[b], acc_sh.at[dsts_v.at[c]], ss[b]).wait()
        if with_cnt:
            pltpu.make_async_copy(ones_v, cnt_sh.at[dsts_v.at[c]], scs[b]).wait()

    pltpu.async_copy(table.at[srcs_v.at[0]], rows[0], sg[0])
    pltpu.async_copy(table.at[srcs_v.at[1]], rows[1], sg[1])
    pltpu.async_copy(table.at[srcs_v.at[2]], rows[2], sg[2])

    def ring(g, carry):
        for b in range(RD):
            c = g * RD + b

            def step(c=c, b=b):
                wait_scatter((b + 3) % RD, c)
                pltpu.async_copy(table.at[srcs_v.at[c + 3]],
                                 rows[(b + 3) % RD], sg[(b + 3) % RD])

            if b < 2:
                @pl.when(g >= 1)
                def _():
                    step()
                @pl.when(g == 0)
                def _():
                    pltpu.async_copy(table.at[srcs_v.at[c + 3]],
                                     rows[(b + 3) % RD], sg[(b + 3) % RD])
            else:
                @pl.when(g < (NCHUNK // RD) - 1)
                def _():
                    step()
                @pl.when(g == (NCHUNK // RD) - 1)
                def _():
                    wait_scatter((b + 3) % RD, c)
            if with_cnt:
                pltpu.async_copy(ones_v, cnt_sh.at[dsts_v.at[c]], scs[b], add=True)
            pltpu.make_async_copy(table.at[srcs_v.at[c]], rows[b], sg[b]).wait()
            pltpu.async_copy(rows[b], acc_sh.at[dsts_v.at[c]], ss[b], add=True)
        return carry

    lax.fori_loop(0, NCHUNK // RD, ring, 0)
    for c in (NCHUNK - 2, NCHUNK - 1):
        wait_scatter(c % RD, c)
    plsc.subcore_barrier()
    pltpu.sync_copy(acc_sh.at[pl.ds(r0, RPT)], out_sum.at[cid].at[pl.ds(r0, RPT)])
    if with_cnt:
        pltpu.sync_copy(cnt_sh.at[pl.ds(r0, RPT)], out_cnt.at[cid].at[pl.ds(r0, RPT)])


@functools.cache
def _sc_agg1():
    return functools.partial(
        pl.kernel,
        mesh=_mesh(),
        compiler_params=pltpu.CompilerParams(use_tc_tiling_on_sc=False),
        out_type=(jax.ShapeDtypeStruct((NC, NP, D), jnp.float32),
                  jax.ShapeDtypeStruct((NC, NP, CW), jnp.float32)),
        scratch_types=(
            [pltpu.VMEM((NCHUNK, CH), jnp.int32),
             pltpu.VMEM((NCHUNK, CH), jnp.int32),
             pltpu.VMEM((CH, CW), jnp.float32)]
            + [pltpu.VMEM((CH, D), jnp.float32)] * RD
            + [pltpu.VMEM_SHARED((NP, D), jnp.float32),
               pltpu.VMEM_SHARED((NP, CW), jnp.float32)]
            + [pltpu.SemaphoreType.DMA] * (3 * RD)
        ),
    )(functools.partial(_sc_agg_body, True))


@functools.cache
def _sc_agg2():
    return functools.partial(
        pl.kernel,
        mesh=_mesh(),
        compiler_params=pltpu.CompilerParams(use_tc_tiling_on_sc=False),
        out_type=jax.ShapeDtypeStruct((NC, NP, D), jnp.float32),
        scratch_types=(
            [pltpu.VMEM((NCHUNK, CH), jnp.int32),
             pltpu.VMEM((NCHUNK, CH), jnp.int32)]
            + [pltpu.VMEM((CH, D), jnp.float32)] * RD
            + [pltpu.VMEM_SHARED((NP, D), jnp.float32)]
            + [pltpu.SemaphoreType.DMA] * (2 * RD)
        ),
    )(functools.partial(_sc_agg_body, False))


def _eo(v, n, w):
    v3 = v.reshape(n // 2, 2, w)
    return v3[:, 0, :], v3[:, 1, :]


def _pair_dot(ve, vo, w):
    return jnp.concatenate(
        [jnp.dot(ve, w, preferred_element_type=jnp.float32),
         jnp.dot(vo, w, preferred_element_type=jnp.float32)], axis=1)


def _proj_body(x_ref, wl_ref, wr_ref, p_ref, xr_ref):
    xe, xo = _eo(x_ref[...], RB, IN_D)
    p_ref[...] = _pair_dot(xe, xo, wl_ref[...])
    xr_ref[...] = _pair_dot(xe, xo, wr_ref[...])


_tc_proj = pl.pallas_call(
    _proj_body,
    grid=(GRID,),
    in_specs=[
        pl.BlockSpec((RB, IN_D), lambda i: (i, 0)),
        pl.BlockSpec((IN_D, D), lambda i: (0, 0)),
        pl.BlockSpec((IN_D, D), lambda i: (0, 0)),
    ],
    out_specs=[
        pl.BlockSpec((PB, 2 * D), lambda i: (i, 0)),
        pl.BlockSpec((PB, 2 * D), lambda i: (i, 0)),
    ],
    out_shape=[jax.ShapeDtypeStruct((NPK, 2 * D), jnp.float32)] * 2,
)


def _packed_inv(c_ref):
    cnt = (c_ref[0] + c_ref[1])[:, 0:1]
    inv = 1.0 / jnp.maximum(cnt, 1.0)
    ie, io = _eo(inv, RB, 1)
    return jnp.concatenate([jnp.broadcast_to(ie, (PB, D)),
                            jnp.broadcast_to(io, (PB, D))], axis=1)


def _mid_body(s_ref, c_ref, xr_ref, b_ref, wl2_ref, wr2_ref, p2_ref, hr2_ref):
    ssum = s_ref[0] + s_ref[1]
    h = jnp.maximum(ssum * _packed_inv(c_ref) + b_ref[...] + xr_ref[...], 0.0)
    p2_ref[...] = _pair_dot(h[:, :D], h[:, D:], wl2_ref[...])
    hr2_ref[...] = _pair_dot(h[:, :D], h[:, D:], wr2_ref[...])


_tc_mid = pl.pallas_call(
    _mid_body,
    grid=(GRID,),
    in_specs=[
        pl.BlockSpec((NC, PB, 2 * D), lambda i: (0, i, 0)),
        pl.BlockSpec((NC, RB, CW), lambda i: (0, i, 0)),
        pl.BlockSpec((PB, 2 * D), lambda i: (i, 0)),
        pl.BlockSpec((1, 2 * D), lambda i: (0, 0)),
        pl.BlockSpec((D, D), lambda i: (0, 0)),
        pl.BlockSpec((D, D), lambda i: (0, 0)),
    ],
    out_specs=[
        pl.BlockSpec((PB, 2 * D), lambda i: (i, 0)),
        pl.BlockSpec((PB, 2 * D), lambda i: (i, 0)),
    ],
    out_shape=[jax.ShapeDtypeStruct((NPK, 2 * D), jnp.float32)] * 2,
)


def _out_body(s_ref, c_ref, hr_ref, b_ref, wc_ref, bc_ref, o_ref):
    ssum = s_ref[0] + s_ref[1]
    h2 = jnp.maximum(ssum * _packed_inv(c_ref) + b_ref[...] + hr_ref[...], 0.0)
    he = h2[:, :D]
    ho = h2[:, D:]
    o_ref[...] = _pair_dot(he, ho, wc_ref[...]) + bc_ref[...]


def _make_tc_out(out_dim):
    return pl.pallas_call(
        _out_body,
        grid=(GRID,),
        in_specs=[
            pl.BlockSpec((NC, PB, 2 * D), lambda i: (0, i, 0)),
            pl.BlockSpec((NC, RB, CW), lambda i: (0, i, 0)),
            pl.BlockSpec((PB, 2 * D), lambda i: (i, 0)),
            pl.BlockSpec((1, 2 * D), lambda i: (0, 0)),
            pl.BlockSpec((D, out_dim), lambda i: (0, 0)),
            pl.BlockSpec((1, 2 * out_dim), lambda i: (0, 0)),
        ],
        out_specs=pl.BlockSpec((PB, 2 * out_dim), lambda i: (i, 0)),
        out_shape=jax.ShapeDtypeStruct((NPK, 2 * out_dim), jnp.float32),
    )


def kernel(x, edge_index, W_l1, b_l1, W_r1, W_l2, b_l2, W_r2, W_c, b_c):
    if edge_index.dtype != jnp.int32:
        edge_index = edge_index.astype(jnp.int32)
    ei = edge_index.reshape(2, NW, NCHUNK, CH)

    b1p = jnp.concatenate([b_l1, b_l1]).reshape(1, 2 * D)
    b2p = jnp.concatenate([b_l2, b_l2]).reshape(1, 2 * D)
    bcp = jnp.concatenate([b_c, b_c]).reshape(1, -1)

    p1p, xr1p = _tc_proj(x, W_l1, W_r1)
    s1, c1 = _sc_agg1()(p1p.reshape(NP, D), ei, _ZROW, _ZCNT, _ONES)
    s1p = s1.reshape(NC, NPK, 2 * D)
    p2p, hr2p = _tc_mid(s1p, c1, xr1p, b1p, W_l2, W_r2)
    s2 = _sc_agg2()(p2p.reshape(NP, D), ei, _ZROW)
    s2p = s2.reshape(NC, NPK, 2 * D)
    outp = _make_tc_out(W_c.shape[1])(s2p, c1, hr2p, b2p, W_c, bcp)
    return outp.reshape(NP, -1)[:N]


# speedup vs baseline: 19.9036x; 1.0290x over previous
import functools

import numpy as np

import jax
import jax.numpy as jnp
from jax import lax
from jax.experimental import pallas as pl
from jax.experimental.pallas import tpu as pltpu
from jax.experimental.pallas import tpu_sc as plsc

N = 10000
E = 320000
IN_D = 128
D = 64
CW = 16

NC, NS = 2, 16
NW = NC * NS
EPW = E // NW
CH = 80
NCHUNK = EPW // CH
NP = 10240
RD = 5
RPT = NP // NS

RB = 2048
PB = RB // 2
GRID = NP // RB
NPK = NP // 2

_ZROW = np.zeros((RPT, D), np.float32)
_ZCNT = np.zeros((RPT, CW), np.float32)
_ONES = np.ones((CH, CW), np.float32)


@functools.cache
def _mesh():
    return plsc.VectorSubcoreMesh(core_axis_name="c", subcore_axis_name="s")


def _sc_agg_body(with_cnt, *refs):
    if with_cnt:
        (table, ei, zrow, zcnt, ones,
         out_sum, out_cnt,
         srcs_v, dsts_v, ones_v) = refs[:10]
        rows = refs[10:10 + RD]
        acc_sh, cnt_sh = refs[10 + RD:12 + RD]
        sg = refs[12 + RD:12 + 2 * RD]
        ss = refs[12 + 2 * RD:12 + 3 * RD]
        scs = refs[12 + 3 * RD:12 + 4 * RD]
    else:
        (table, ei, zrow,
         out_sum,
         srcs_v, dsts_v) = refs[:6]
        rows = refs[6:6 + RD]
        acc_sh = refs[6 + RD]
        sg = refs[7 + RD:7 + 2 * RD]
        ss = refs[7 + 2 * RD:7 + 3 * RD]
        scs = None
    cid = lax.axis_index("c")
    sid = lax.axis_index("s")
    wid = cid * NS + sid
    r0 = sid * RPT
    hs = [pltpu.async_copy(zrow, acc_sh.at[pl.ds(r0, RPT)], sg[0]),
          pltpu.async_copy(ei.at[0, wid], srcs_v, sg[1]),
          pltpu.async_copy(ei.at[1, wid], dsts_v, sg[2])]
    if with_cnt:
        hs.append(pltpu.async_copy(zcnt, cnt_sh.at[pl.ds(r0, RPT)], sg[3]))
        hs.append(pltpu.async_copy(ones, ones_v, sg[4]))
    for h in hs:
        h.wait()
    plsc.subcore_barrier()

    def wait_scatter(b, c):
        pltpu.make_async_copy(rows[b], acc_sh.at[dsts_v.at[c]], ss[b]).wait()
        if with_cnt:
            pltpu.make_async_copy(ones_v, cnt_sh.at[dsts_v.at[c]], scs[b]).wait()

    pltpu.async_copy(table.at[srcs_v.at[0]], rows[0], sg[0])
    pltpu.async_copy(table.at[srcs_v.at[1]], rows[1], sg[1])
    pltpu.async_copy(table.at[srcs_v.at[2]], rows[2], sg[2])

    def ring(g, carry):
        for b in range(RD):
            c = g * RD + b

            def step(c=c, b=b):
                wait_scatter((b + 3) % RD, c)
                pltpu.async_copy(table.at[srcs_v.at[c + 3]],
                                 rows[(b + 3) % RD], sg[(b + 3) % RD])

            if b < 2:
                @pl.when(g >= 1)
                def _():
                    step()
                @pl.when(g == 0)
                def _():
                    pltpu.async_copy(table.at[srcs_v.at[c + 3]],
                                     rows[(b + 3) % RD], sg[(b + 3) % RD])
            else:
                @pl.when(g < (NCHUNK // RD) - 1)
                def _():
                    step()
                @pl.when(g == (NCHUNK // RD) - 1)
                def _():
                    wait_scatter((b + 3) % RD, c)
            if with_cnt:
                pltpu.async_copy(ones_v, cnt_sh.at[dsts_v.at[c]], scs[b], add=True)
            pltpu.make_async_copy(table.at[srcs_v.at[c]], rows[b], sg[b]).wait()
            pltpu.async_copy(rows[b], acc_sh.at[dsts_v.at[c]], ss[b], add=True)
        return carry

    lax.fori_loop(0, NCHUNK // RD, ring, 0)
    for c in (NCHUNK - 2, NCHUNK - 1):
        wait_scatter(c % RD, c)
    plsc.subcore_barrier()
    ho = [pltpu.async_copy(acc_sh.at[pl.ds(r0, RPT)],
                           out_sum.at[cid].at[pl.ds(r0, RPT)], sg[0])]
    if with_cnt:
        ho.append(pltpu.async_copy(cnt_sh.at[pl.ds(r0, RPT)],
                                   out_cnt.at[cid].at[pl.ds(r0, RPT)], sg[1]))
    for h in ho:
        h.wait()


@functools.cache
def _sc_agg1():
    return functools.partial(
        pl.kernel,
        mesh=_mesh(),
        compiler_params=pltpu.CompilerParams(use_tc_tiling_on_sc=False),
        out_type=(jax.ShapeDtypeStruct((NC, NP, D), jnp.float32),
                  jax.ShapeDtypeStruct((NC, NP, CW), jnp.float32)),
        scratch_types=(
            [pltpu.VMEM((NCHUNK, CH), jnp.int32),
             pltpu.VMEM((NCHUNK, CH), jnp.int32),
             pltpu.VMEM((CH, CW), jnp.float32)]
            + [pltpu.VMEM((CH, D), jnp.float32)] * RD
            + [pltpu.VMEM_SHARED((NP, D), jnp.float32),
               pltpu.VMEM_SHARED((NP, CW), jnp.float32)]
            + [pltpu.SemaphoreType.DMA] * (3 * RD)
        ),
    )(functools.partial(_sc_agg_body, True))


@functools.cache
def _sc_agg2():
    return functools.partial(
        pl.kernel,
        mesh=_mesh(),
        compiler_params=pltpu.CompilerParams(use_tc_tiling_on_sc=False),
        out_type=jax.ShapeDtypeStruct((NC, NP, D), jnp.float32),
        scratch_types=(
            [pltpu.VMEM((NCHUNK, CH), jnp.int32),
             pltpu.VMEM((NCHUNK, CH), jnp.int32)]
            + [pltpu.VMEM((CH, D), jnp.float32)] * RD
            + [pltpu.VMEM_SHARED((NP, D), jnp.float32)]
            + [pltpu.SemaphoreType.DMA] * (2 * RD)
        ),
    )(functools.partial(_sc_agg_body, False))


def _eo(v, n, w):
    v3 = v.reshape(n // 2, 2, w)
    return v3[:, 0, :], v3[:, 1, :]


def _pair_dot(ve, vo, w):
    return jnp.concatenate(
        [jnp.dot(ve, w, preferred_element_type=jnp.float32),
         jnp.dot(vo, w, preferred_element_type=jnp.float32)], axis=1)


def _proj_body(x_ref, wl_ref, wr_ref, p_ref, xr_ref):
    xe, xo = _eo(x_ref[...], RB, IN_D)
    p_ref[...] = _pair_dot(xe, xo, wl_ref[...])
    xr_ref[...] = _pair_dot(xe, xo, wr_ref[...])


_tc_proj = pl.pallas_call(
    _proj_body,
    grid=(GRID,),
    in_specs=[
        pl.BlockSpec((RB, IN_D), lambda i: (i, 0)),
        pl.BlockSpec((IN_D, D), lambda i: (0, 0)),
        pl.BlockSpec((IN_D, D), lambda i: (0, 0)),
    ],
    out_specs=[
        pl.BlockSpec((PB, 2 * D), lambda i: (i, 0)),
        pl.BlockSpec((PB, 2 * D), lambda i: (i, 0)),
    ],
    out_shape=[jax.ShapeDtypeStruct((NPK, 2 * D), jnp.float32)] * 2,
)


def _packed_inv(c_ref):
    cnt = (c_ref[0] + c_ref[1])[:, 0:1]
    inv = 1.0 / jnp.maximum(cnt, 1.0)
    ie, io = _eo(inv, RB, 1)
    return jnp.concatenate([jnp.broadcast_to(ie, (PB, D)),
                            jnp.broadcast_to(io, (PB, D))], axis=1)


def _mid_body(s_ref, c_ref, xr_ref, b_ref, wl2_ref, wr2_ref, p2_ref, hr2_ref):
    ssum = s_ref[0] + s_ref[1]
    h = jnp.maximum(ssum * _packed_inv(c_ref) + b_ref[...] + xr_ref[...], 0.0)
    p2_ref[...] = _pair_dot(h[:, :D], h[:, D:], wl2_ref[...])
    hr2_ref[...] = _pair_dot(h[:, :D], h[:, D:], wr2_ref[...])


_tc_mid = pl.pallas_call(
    _mid_body,
    grid=(GRID,),
    in_specs=[
        pl.BlockSpec((NC, PB, 2 * D), lambda i: (0, i, 0)),
        pl.BlockSpec((NC, RB, CW), lambda i: (0, i, 0)),
        pl.BlockSpec((PB, 2 * D), lambda i: (i, 0)),
        pl.BlockSpec((1, 2 * D), lambda i: (0, 0)),
        pl.BlockSpec((D, D), lambda i: (0, 0)),
        pl.BlockSpec((D, D), lambda i: (0, 0)),
    ],
    out_specs=[
        pl.BlockSpec((PB, 2 * D), lambda i: (i, 0)),
        pl.BlockSpec((PB, 2 * D), lambda i: (i, 0)),
    ],
    out_shape=[jax.ShapeDtypeStruct((NPK, 2 * D), jnp.float32)] * 2,
)


def _out_body(s_ref, c_ref, hr_ref, b_ref, wc_ref, bc_ref, o_ref):
    ssum = s_ref[0] + s_ref[1]
    h2 = jnp.maximum(ssum * _packed_inv(c_ref) + b_ref[...] + hr_ref[...], 0.0)
    he = h2[:, :D]
    ho = h2[:, D:]
    o_ref[...] = _pair_dot(he, ho, wc_ref[...]) + bc_ref[...]


def _make_tc_out(out_dim):
    return pl.pallas_call(
        _out_body,
        grid=(GRID,),
        in_specs=[
            pl.BlockSpec((NC, PB, 2 * D), lambda i: (0, i, 0)),
            pl.BlockSpec((NC, RB, CW), lambda i: (0, i, 0)),
            pl.BlockSpec((PB, 2 * D), lambda i: (i, 0)),
            pl.BlockSpec((1, 2 * D), lambda i: (0, 0)),
            pl.BlockSpec((D, out_dim), lambda i: (0, 0)),
            pl.BlockSpec((1, 2 * out_dim), lambda i: (0, 0)),
        ],
        out_specs=pl.BlockSpec((PB, 2 * out_dim), lambda i: (i, 0)),
        out_shape=jax.ShapeDtypeStruct((NPK, 2 * out_dim), jnp.float32),
    )


def kernel(x, edge_index, W_l1, b_l1, W_r1, W_l2, b_l2, W_r2, W_c, b_c):
    if edge_index.dtype != jnp.int32:
        edge_index = edge_index.astype(jnp.int32)
    ei = edge_index.reshape(2, NW, NCHUNK, CH)

    b1p = jnp.concatenate([b_l1, b_l1]).reshape(1, 2 * D)
    b2p = jnp.concatenate([b_l2, b_l2]).reshape(1, 2 * D)
    bcp = jnp.concatenate([b_c, b_c]).reshape(1, -1)

    p1p, xr1p = _tc_proj(x, W_l1, W_r1)
    s1, c1 = _sc_agg1()(p1p.reshape(NP, D), ei, _ZROW, _ZCNT, _ONES)
    s1p = s1.reshape(NC, NPK, 2 * D)
    p2p, hr2p = _tc_mid(s1p, c1, xr1p, b1p, W_l2, W_r2)
    s2 = _sc_agg2()(p2p.reshape(NP, D), ei, _ZROW)
    s2p = s2.reshape(NC, NPK, 2 * D)
    outp = _make_tc_out(W_c.shape[1])(s2p, c1, hr2p, b2p, W_c, bcp)
    return outp.reshape(NP, -1)[:N]


# speedup vs baseline: 21.6307x; 1.0868x over previous
import functools

import numpy as np

import jax
import jax.numpy as jnp
from jax import lax
from jax.experimental import pallas as pl
from jax.experimental.pallas import tpu as pltpu
from jax.experimental.pallas import tpu_sc as plsc

N = 10000
E = 320000
IN_D = 128
D = 64
CW = 16

NC, NS = 2, 16
NW = NC * NS
EPW = E // NW
CH = 80
NCHUNK = EPW // CH
NP = 10240
RD = 5
RPT = NP // NS

RB = 2048
PB = RB // 2
GRID = NP // RB
NPK = NP // 2

_ZROW = np.zeros((RPT, D), np.float32)

_MEXP = np.zeros((128, 512), np.float32)
for _r in range(4):
    _MEXP[32 * _r, 128 * _r:128 * _r + 64] = 1.0
    _MEXP[32 * _r + 16, 128 * _r + 64:128 * _r + 128] = 1.0
_ZCNT = np.zeros((RPT, CW), np.float32)
_ONES = np.ones((CH, CW), np.float32)


@functools.cache
def _mesh():
    return plsc.VectorSubcoreMesh(core_axis_name="c", subcore_axis_name="s")


def _sc_agg_body(with_cnt, *refs):
    if with_cnt:
        (table, ei, zrow, zcnt, ones,
         out_sum, out_cnt,
         srcs_v, dsts_v, ones_v) = refs[:10]
        rows = refs[10:10 + RD]
        acc_sh, cnt_sh = refs[10 + RD:12 + RD]
        sg = refs[12 + RD:12 + 2 * RD]
        ss = refs[12 + 2 * RD:12 + 3 * RD]
        scs = refs[12 + 3 * RD:12 + 4 * RD]
    else:
        (table, ei, zrow,
         out_sum,
         srcs_v, dsts_v) = refs[:6]
        rows = refs[6:6 + RD]
        acc_sh = refs[6 + RD]
        sg = refs[7 + RD:7 + 2 * RD]
        ss = refs[7 + 2 * RD:7 + 3 * RD]
        scs = None
    cid = lax.axis_index("c")
    sid = lax.axis_index("s")
    wid = cid * NS + sid
    r0 = sid * RPT
    hs = [pltpu.async_copy(zrow, acc_sh.at[pl.ds(r0, RPT)], sg[0]),
          pltpu.async_copy(ei.at[0, wid], srcs_v, sg[1]),
          pltpu.async_copy(ei.at[1, wid], dsts_v, sg[2])]
    if with_cnt:
        hs.append(pltpu.async_copy(zcnt, cnt_sh.at[pl.ds(r0, RPT)], sg[3]))
        hs.append(pltpu.async_copy(ones, ones_v, sg[4]))
    for h in hs:
        h.wait()
    plsc.subcore_barrier()

    def wait_scatter(b, c):
        pltpu.make_async_copy(rows[b], acc_sh.at[dsts_v.at[c]], ss[b]).wait()
        if with_cnt:
            pltpu.make_async_copy(ones_v, cnt_sh.at[dsts_v.at[c]], scs[b]).wait()

    pltpu.async_copy(table.at[srcs_v.at[0]], rows[0], sg[0])
    pltpu.async_copy(table.at[srcs_v.at[1]], rows[1], sg[1])
    pltpu.async_copy(table.at[srcs_v.at[2]], rows[2], sg[2])

    def ring(g, carry):
        for b in range(RD):
            c = g * RD + b

            def step(c=c, b=b):
                wait_scatter((b + 3) % RD, c)
                pltpu.async_copy(table.at[srcs_v.at[c + 3]],
                                 rows[(b + 3) % RD], sg[(b + 3) % RD])

            if b < 2:
                @pl.when(g >= 1)
                def _():
                    step()
                @pl.when(g == 0)
                def _():
                    pltpu.async_copy(table.at[srcs_v.at[c + 3]],
                                     rows[(b + 3) % RD], sg[(b + 3) % RD])
            else:
                @pl.when(g < (NCHUNK // RD) - 1)
                def _():
                    step()
                @pl.when(g == (NCHUNK // RD) - 1)
                def _():
                    wait_scatter((b + 3) % RD, c)
            if with_cnt:
                pltpu.async_copy(ones_v, cnt_sh.at[dsts_v.at[c]], scs[b], add=True)
            pltpu.make_async_copy(table.at[srcs_v.at[c]], rows[b], sg[b]).wait()
            pltpu.async_copy(rows[b], acc_sh.at[dsts_v.at[c]], ss[b], add=True)
        return carry

    lax.fori_loop(0, NCHUNK // RD, ring, 0)
    for c in (NCHUNK - 2, NCHUNK - 1):
        wait_scatter(c % RD, c)
    plsc.subcore_barrier()
    ho = [pltpu.async_copy(acc_sh.at[pl.ds(r0, RPT)],
                           out_sum.at[cid].at[pl.ds(r0, RPT)], sg[0])]
    if with_cnt:
        ho.append(pltpu.async_copy(cnt_sh.at[pl.ds(r0, RPT)],
                                   out_cnt.at[cid].at[pl.ds(r0, RPT)], sg[1]))
    for h in ho:
        h.wait()


@functools.cache
def _sc_agg1():
    return functools.partial(
        pl.kernel,
        mesh=_mesh(),
        compiler_params=pltpu.CompilerParams(use_tc_tiling_on_sc=False),
        out_type=(jax.ShapeDtypeStruct((NC, NP, D), jnp.float32),
                  jax.ShapeDtypeStruct((NC, NP, CW), jnp.float32)),
        scratch_types=(
            [pltpu.VMEM((NCHUNK, CH), jnp.int32),
             pltpu.VMEM((NCHUNK, CH), jnp.int32),
             pltpu.VMEM((CH, CW), jnp.float32)]
            + [pltpu.VMEM((CH, D), jnp.float32)] * RD
            + [pltpu.VMEM_SHARED((NP, D), jnp.float32),
               pltpu.VMEM_SHARED((NP, CW), jnp.float32)]
            + [pltpu.SemaphoreType.DMA] * (3 * RD)
        ),
    )(functools.partial(_sc_agg_body, True))


@functools.cache
def _sc_agg2():
    return functools.partial(
        pl.kernel,
        mesh=_mesh(),
        compiler_params=pltpu.CompilerParams(use_tc_tiling_on_sc=False),
        out_type=jax.ShapeDtypeStruct((NC, NP, D), jnp.float32),
        scratch_types=(
            [pltpu.VMEM((NCHUNK, CH), jnp.int32),
             pltpu.VMEM((NCHUNK, CH), jnp.int32)]
            + [pltpu.VMEM((CH, D), jnp.float32)] * RD
            + [pltpu.VMEM_SHARED((NP, D), jnp.float32)]
            + [pltpu.SemaphoreType.DMA] * (2 * RD)
        ),
    )(functools.partial(_sc_agg_body, False))


def _eo(v, n, w):
    v3 = v.reshape(n // 2, 2, w)
    return v3[:, 0, :], v3[:, 1, :]


def _pair_dot(ve, vo, w):
    return jnp.concatenate(
        [jnp.dot(ve, w, preferred_element_type=jnp.float32),
         jnp.dot(vo, w, preferred_element_type=jnp.float32)], axis=1)


def _proj_body(x_ref, wl_ref, wr_ref, p_ref, xr_ref):
    xe, xo = _eo(x_ref[...], RB, IN_D)
    p_ref[...] = _pair_dot(xe, xo, wl_ref[...])
    xr_ref[...] = _pair_dot(xe, xo, wr_ref[...])


_tc_proj = pl.pallas_call(
    _proj_body,
    grid=(GRID,),
    in_specs=[
        pl.BlockSpec((RB, IN_D), lambda i: (i, 0)),
        pl.BlockSpec((IN_D, D), lambda i: (0, 0)),
        pl.BlockSpec((IN_D, D), lambda i: (0, 0)),
    ],
    out_specs=[
        pl.BlockSpec((PB, 2 * D), lambda i: (i, 0)),
        pl.BlockSpec((PB, 2 * D), lambda i: (i, 0)),
    ],
    out_shape=[jax.ShapeDtypeStruct((NPK, 2 * D), jnp.float32)] * 2,
)


CB = RB // 8


def _mid_body(s_ref, c_ref, xr_ref, b_ref, wl2_ref, wr2_ref, me_ref,
              p2_ref, hr2_ref, invp_ref):
    csum = c_ref[0] + c_ref[1]
    inv = 1.0 / jnp.maximum(csum, 1.0)
    invp = jnp.dot(inv, me_ref[...],
                   preferred_element_type=jnp.float32).reshape(PB, 2 * D)
    invp_ref[...] = invp
    ssum = s_ref[0] + s_ref[1]
    h = jnp.maximum(ssum * invp + b_ref[...] + xr_ref[...], 0.0)
    p2_ref[...] = _pair_dot(h[:, :D], h[:, D:], wl2_ref[...])
    hr2_ref[...] = _pair_dot(h[:, :D], h[:, D:], wr2_ref[...])


_tc_mid = pl.pallas_call(
    _mid_body,
    grid=(GRID,),
    in_specs=[
        pl.BlockSpec((NC, PB, 2 * D), lambda i: (0, i, 0)),
        pl.BlockSpec((NC, CB, 128), lambda i: (0, i, 0)),
        pl.BlockSpec((PB, 2 * D), lambda i: (i, 0)),
        pl.BlockSpec((1, 2 * D), lambda i: (0, 0)),
        pl.BlockSpec((D, D), lambda i: (0, 0)),
        pl.BlockSpec((D, D), lambda i: (0, 0)),
        pl.BlockSpec((128, 512), lambda i: (0, 0)),
    ],
    out_specs=[
        pl.BlockSpec((PB, 2 * D), lambda i: (i, 0)),
        pl.BlockSpec((PB, 2 * D), lambda i: (i, 0)),
        pl.BlockSpec((PB, 2 * D), lambda i: (i, 0)),
    ],
    out_shape=[jax.ShapeDtypeStruct((NPK, 2 * D), jnp.float32)] * 3,
)


def _out_body(s_ref, invp_ref, hr_ref, b_ref, wc_ref, bc_ref, o_ref):
    ssum = s_ref[0] + s_ref[1]
    h2 = jnp.maximum(ssum * invp_ref[...] + b_ref[...] + hr_ref[...], 0.0)
    he = h2[:, :D]
    ho = h2[:, D:]
    o_ref[...] = _pair_dot(he, ho, wc_ref[...]) + bc_ref[...]


def _make_tc_out(out_dim):
    return pl.pallas_call(
        _out_body,
        grid=(GRID,),
        in_specs=[
            pl.BlockSpec((NC, PB, 2 * D), lambda i: (0, i, 0)),
            pl.BlockSpec((PB, 2 * D), lambda i: (i, 0)),
            pl.BlockSpec((PB, 2 * D), lambda i: (i, 0)),
            pl.BlockSpec((1, 2 * D), lambda i: (0, 0)),
            pl.BlockSpec((D, out_dim), lambda i: (0, 0)),
            pl.BlockSpec((1, 2 * out_dim), lambda i: (0, 0)),
        ],
        out_specs=pl.BlockSpec((PB, 2 * out_dim), lambda i: (i, 0)),
        out_shape=jax.ShapeDtypeStruct((NPK, 2 * out_dim), jnp.float32),
    )


def kernel(x, edge_index, W_l1, b_l1, W_r1, W_l2, b_l2, W_r2, W_c, b_c):
    if edge_index.dtype != jnp.int32:
        edge_index = edge_index.astype(jnp.int32)
    ei = edge_index.reshape(2, NW, NCHUNK, CH)

    b1p = jnp.concatenate([b_l1, b_l1]).reshape(1, 2 * D)
    b2p = jnp.concatenate([b_l2, b_l2]).reshape(1, 2 * D)
    bcp = jnp.concatenate([b_c, b_c]).reshape(1, -1)

    p1p, xr1p = _tc_proj(x, W_l1, W_r1)
    s1, c1 = _sc_agg1()(p1p.reshape(NP, D), ei, _ZROW, _ZCNT, _ONES)
    s1p = s1.reshape(NC, NPK, 2 * D)
    c1l = c1.reshape(NC, NP * CW // 128, 128)
    p2p, hr2p, invp = _tc_mid(s1p, c1l, xr1p, b1p, W_l2, W_r2, _MEXP)
    s2 = _sc_agg2()(p2p.reshape(NP, D), ei, _ZROW)
    s2p = s2.reshape(NC, NPK, 2 * D)
    outp = _make_tc_out(W_c.shape[1])(s2p, invp, hr2p, b2p, W_c, bcp)
    return outp.reshape(NP, -1)[:N]
